# baseline (speedup 1.0000x reference)
import jax
import jax.numpy as jnp
from jax import lax
from jax.experimental import pallas as pl
from jax.experimental.pallas import tpu as pltpu
from jax.experimental.pallas import tpu_sc as plsc

M = 1_000_000
D = 32
LD = 16
B = 16384
MP = 128

NC = 2
NS = 16
NW = NC * NS
BPW = B // NW
CHUNK = 128
NCHUNK = BPW // CHUNK

_sc_mesh = plsc.VectorSubcoreMesh(core_axis_name="c", subcore_axis_name="s")
_sc_params = pltpu.CompilerParams(use_tc_tiling_on_sc=False,
                                  needs_layout_passes=False)


def _wid():
    return lax.axis_index("s") * NC + lax.axis_index("c")


@pl.kernel(
    out_type=jax.ShapeDtypeStruct((B, D), jnp.float32),
    mesh=_sc_mesh,
    compiler_params=_sc_params,
    scratch_types=[
        pltpu.VMEM((NCHUNK, CHUNK), jnp.int32),
        pltpu.VMEM((BPW, MP), jnp.float32),
        pltpu.SemaphoreType.DMA,
    ],
)
def _sc_gather(buf_ref, idx_hbm, out_hbm, idx_v, rows_v, sem):
    w = _wid()
    pltpu.sync_copy(idx_hbm.at[pl.ds(w * NCHUNK, NCHUNK)], idx_v)
    for j in range(NCHUNK):
        pltpu.async_copy(
            buf_ref.at[idx_v.at[j]], rows_v.at[pl.ds(j * CHUNK, CHUNK)], sem
        )
    for j in range(NCHUNK):
        pltpu.make_async_copy(
            buf_ref.at[idx_v.at[j]], rows_v.at[pl.ds(j * CHUNK, CHUNK)], sem
        ).wait()
    pltpu.sync_copy(rows_v.at[:, pl.ds(0, D)], out_hbm.at[pl.ds(w * BPW, BPW)])


@pl.kernel(
    mesh=_sc_mesh,
    compiler_params=_sc_params,
    scratch_types=[
        pltpu.VMEM((NCHUNK, CHUNK), jnp.int32),
        pltpu.VMEM((BPW, MP), jnp.float32),
        pltpu.SemaphoreType.DMA,
    ],
)
def _sc_scatter(buf_ref, idx_hbm, recon_hbm, idx_v, rows_v, sem):
    w = _wid()
    pltpu.sync_copy(idx_hbm.at[pl.ds(w * NCHUNK, NCHUNK)], idx_v)
    pltpu.sync_copy(recon_hbm.at[pl.ds(w * BPW, BPW)], rows_v.at[:, pl.ds(0, D)])
    for j in range(NCHUNK):
        pltpu.async_copy(
            rows_v.at[pl.ds(j * CHUNK, CHUNK)], buf_ref.at[idx_v.at[j]], sem
        )
    for j in range(NCHUNK):
        pltpu.make_async_copy(
            rows_v.at[pl.ds(j * CHUNK, CHUNK)], buf_ref.at[idx_v.at[j]], sem
        ).wait()


def _vae_body(rows_ref, val_ref, eps_ref, wmu_ref, bmu_ref, wlv_ref, blv_ref,
              wdec_ref, bdec_ref, out_ref):
    h = rows_ref[...] + val_ref[...]
    mu = jnp.dot(h, wmu_ref[...], preferred_element_type=jnp.float32,
                 precision=lax.Precision.HIGHEST) + bmu_ref[...]
    logvar = jnp.dot(h, wlv_ref[...], preferred_element_type=jnp.float32,
                     precision=lax.Precision.HIGHEST) + blv_ref[...]
    z = mu + jnp.exp(0.5 * logvar) * eps_ref[...]
    out_ref[...] = jnp.dot(z, wdec_ref[...], preferred_element_type=jnp.float32,
                           precision=lax.Precision.HIGHEST) + bdec_ref[...]


VAE_BLK = 2048

_vae = pl.pallas_call(
    _vae_body,
    grid=(B // VAE_BLK,),
    in_specs=[
        pl.BlockSpec((VAE_BLK, D), lambda i: (i, 0)),
        pl.BlockSpec((VAE_BLK, D), lambda i: (i, 0)),
        pl.BlockSpec((VAE_BLK, LD), lambda i: (i, 0)),
        pl.BlockSpec((D, LD), lambda i: (0, 0)),
        pl.BlockSpec((1, LD), lambda i: (0, 0)),
        pl.BlockSpec((D, LD), lambda i: (0, 0)),
        pl.BlockSpec((1, LD), lambda i: (0, 0)),
        pl.BlockSpec((LD, D), lambda i: (0, 0)),
        pl.BlockSpec((1, D), lambda i: (0, 0)),
    ],
    out_specs=pl.BlockSpec((VAE_BLK, D), lambda i: (i, 0)),
    out_shape=jax.ShapeDtypeStruct((B, D), jnp.float32),
)


WCOL = 8192


def _widen_body(int_ref, out_ref):
    out_ref[...] = jnp.pad(int_ref[...].T, ((0, 0), (0, MP - D)))


_widen = pl.pallas_call(
    _widen_body,
    grid=((M + WCOL - 1) // WCOL,),
    in_specs=[pl.BlockSpec((D, WCOL), lambda i: (0, i))],
    out_specs=pl.BlockSpec((WCOL, MP), lambda i: (i, 0)),
    out_shape=jax.ShapeDtypeStruct((M, MP), jnp.float32),
)


def kernel(mem, idx, val, eps, W_mu, b_mu, W_lv, b_lv, W_dec, b_dec):
    idx2d = idx.astype(jnp.int32).reshape(B // CHUNK, CHUNK)
    buf_ref = jax.new_ref(_widen(mem.T))
    rows = _sc_gather(buf_ref, idx2d)
    recon = _vae(rows, val, eps, W_mu, b_mu.reshape(1, LD), W_lv,
                 b_lv.reshape(1, LD), W_dec, b_dec.reshape(1, D))
    _sc_scatter(buf_ref, idx2d, recon)
    return buf_ref[...][:, :D]


# speedup vs baseline: 1.0855x; 1.0855x over previous
import jax
import jax.numpy as jnp
from jax import lax
from jax.experimental import pallas as pl
from jax.experimental.pallas import tpu as pltpu
from jax.experimental.pallas import tpu_sc as plsc

M = 1_000_000
D = 32
LD = 16
B = 16384
MP = 128

NC = 2
NS = 16
NW = NC * NS
BPW = B // NW
CHUNK = 128
NCHUNK = BPW // CHUNK

_sc_mesh = plsc.VectorSubcoreMesh(core_axis_name="c", subcore_axis_name="s")
_sc_params = pltpu.CompilerParams(use_tc_tiling_on_sc=False,
                                  needs_layout_passes=False)


def _wid():
    return lax.axis_index("s") * NC + lax.axis_index("c")


@pl.kernel(
    out_type=jax.ShapeDtypeStruct((B, MP), jnp.float32),
    mesh=_sc_mesh,
    compiler_params=_sc_params,
    scratch_types=[
        pltpu.VMEM((NCHUNK, CHUNK), jnp.int32),
        pltpu.VMEM((BPW, MP), jnp.float32),
        pltpu.SemaphoreType.DMA,
    ],
)
def _sc_gather(buf_ref, idx_hbm, out_hbm, idx_v, rows_v, sem):
    w = _wid()
    pltpu.sync_copy(idx_hbm.at[pl.ds(w * NCHUNK, NCHUNK)], idx_v)
    for j in range(NCHUNK):
        pltpu.async_copy(
            buf_ref.at[idx_v.at[j]], rows_v.at[pl.ds(j * CHUNK, CHUNK)], sem
        )
    for j in range(NCHUNK):
        pltpu.make_async_copy(
            buf_ref.at[idx_v.at[j]], rows_v.at[pl.ds(j * CHUNK, CHUNK)], sem
        ).wait()
    pltpu.sync_copy(rows_v, out_hbm.at[pl.ds(w * BPW, BPW)])


@pl.kernel(
    mesh=_sc_mesh,
    compiler_params=_sc_params,
    scratch_types=[
        pltpu.VMEM((NCHUNK, CHUNK), jnp.int32),
        pltpu.VMEM((BPW, MP), jnp.float32),
        pltpu.SemaphoreType.DMA,
    ],
)
def _sc_scatter(buf_ref, idx_hbm, recon_hbm, idx_v, rows_v, sem):
    w = _wid()
    pltpu.sync_copy(idx_hbm.at[pl.ds(w * NCHUNK, NCHUNK)], idx_v)
    pltpu.sync_copy(recon_hbm.at[pl.ds(w * BPW, BPW)], rows_v)
    for j in range(NCHUNK):
        pltpu.async_copy(
            rows_v.at[pl.ds(j * CHUNK, CHUNK)], buf_ref.at[idx_v.at[j]], sem
        )
    for j in range(NCHUNK):
        pltpu.make_async_copy(
            rows_v.at[pl.ds(j * CHUNK, CHUNK)], buf_ref.at[idx_v.at[j]], sem
        ).wait()


def _vae_body(rows_ref, val_ref, eps_ref, wmu_ref, bmu_ref, wlv_ref, blv_ref,
              wdec_ref, bdec_ref, out_ref):
    h = rows_ref[:, :D] + val_ref[...]
    mu = jnp.dot(h, wmu_ref[...], preferred_element_type=jnp.float32,
                 precision=lax.Precision.HIGHEST) + bmu_ref[...]
    logvar = jnp.dot(h, wlv_ref[...], preferred_element_type=jnp.float32,
                     precision=lax.Precision.HIGHEST) + blv_ref[...]
    z = mu + jnp.exp(0.5 * logvar) * eps_ref[...]
    recon = jnp.dot(z, wdec_ref[...], preferred_element_type=jnp.float32,
                    precision=lax.Precision.HIGHEST) + bdec_ref[...]
    out_ref[...] = jnp.pad(recon, ((0, 0), (0, MP - D)))


VAE_BLK = 8192

_vae = pl.pallas_call(
    _vae_body,
    grid=(B // VAE_BLK,),
    in_specs=[
        pl.BlockSpec((VAE_BLK, MP), lambda i: (i, 0)),
        pl.BlockSpec((VAE_BLK, D), lambda i: (i, 0)),
        pl.BlockSpec((VAE_BLK, LD), lambda i: (i, 0)),
        pl.BlockSpec((D, LD), lambda i: (0, 0)),
        pl.BlockSpec((1, LD), lambda i: (0, 0)),
        pl.BlockSpec((D, LD), lambda i: (0, 0)),
        pl.BlockSpec((1, LD), lambda i: (0, 0)),
        pl.BlockSpec((LD, D), lambda i: (0, 0)),
        pl.BlockSpec((1, D), lambda i: (0, 0)),
    ],
    out_specs=pl.BlockSpec((VAE_BLK, MP), lambda i: (i, 0)),
    out_shape=jax.ShapeDtypeStruct((B, MP), jnp.float32),
)


WCOL = 16384


def _widen_body(int_ref, out_ref):
    out_ref[...] = jnp.pad(int_ref[...].T, ((0, 0), (0, MP - D)))


_widen = pl.pallas_call(
    _widen_body,
    grid=((M + WCOL - 1) // WCOL,),
    in_specs=[pl.BlockSpec((D, WCOL), lambda i: (0, i))],
    out_specs=pl.BlockSpec((WCOL, MP), lambda i: (i, 0)),
    out_shape=jax.ShapeDtypeStruct((M, MP), jnp.float32),
)


def kernel(mem, idx, val, eps, W_mu, b_mu, W_lv, b_lv, W_dec, b_dec):
    idx2d = idx.astype(jnp.int32).reshape(B // CHUNK, CHUNK)
    buf_ref = jax.new_ref(_widen(mem.T))
    rows = _sc_gather(buf_ref, idx2d)
    recon = _vae(rows, val, eps, W_mu, b_mu.reshape(1, LD), W_lv,
                 b_lv.reshape(1, LD), W_dec, b_dec.reshape(1, D))
    _sc_scatter(buf_ref, idx2d, recon)
    return buf_ref[...][:, :D]
